# baseline (speedup 1.0000x reference)
import functools

import jax
import jax.numpy as jnp
from jax import lax
from jax.experimental import pallas as pl
from jax.experimental.pallas import tpu as pltpu
from jax.experimental.pallas import tpu_sc as plsc

N_DATA = 100000
N_DIM = 128
BSZ = 256
K = 2048
T = 0.07
M = 0.5
EPS = 1e-12

RB = 2048
NBLK = 49
NPAD = RB * NBLK
SLAB = NPAD // 128
NW = 32
BPW = BSZ // NW
NCH = 24
PADW = NCH * 128


def _mm_body(xz_ref, mem_ref, f_ref):
    a = mem_ref[...]
    d = lax.dot_general(xz_ref[...], a.astype(jnp.bfloat16),
                        (((1,), (1,)), ((), ())),
                        preferred_element_type=jnp.float32)
    def rne(v32):
        return (v32 + 0x7FFF + ((v32 >> 16) & 1)) & jnp.uint32(0xFFFF0000)

    xb = rne(lax.bitcast_convert_type(d[:BSZ], jnp.uint32))
    zb = rne(lax.bitcast_convert_type(d[BSZ:], jnp.uint32))
    packed = lax.bitcast_convert_type((xb >> 16) | zb, jnp.int32)
    f_ref[0] = packed


def _logits_all(memory, xz):
    return pl.pallas_call(
        _mm_body,
        grid=(NBLK,),
        in_specs=[
            pl.BlockSpec((2 * BSZ, N_DIM), lambda i: (0, 0)),
            pl.BlockSpec((RB, N_DIM), lambda i: (i, 0)),
        ],
        out_specs=pl.BlockSpec((1, BSZ, RB), lambda i: (i, 0, 0)),
        out_shape=jax.ShapeDtypeStruct((NBLK, BSZ, RB), jnp.int32),
    )(xz, memory)


def _sc_gather(ftp_hbm, g_hbm, lxz_hbm, row_v, g_v, ssem):
    wid = lax.axis_index("s") * 2 + lax.axis_index("c")
    b0 = wid * BPW

    @pl.loop(0, BPW)
    def _per_b(lb):
        b = b0 + lb
        cp1 = pltpu.async_copy(
            ftp_hbm.at[:, b, pl.ds(0, RB // 2)],
            row_v.at[:, pl.ds(0, RB // 2)], ssem)
        cp2 = pltpu.async_copy(
            ftp_hbm.at[:, b, pl.ds(RB // 2, RB // 2)],
            row_v.at[:, pl.ds(RB // 2, RB // 2)], ssem)
        pltpu.sync_copy(g_hbm.at[b], g_v)
        cp1.wait()
        cp2.wait()
        for t in range(NCH):
            for s in range(8):
                g16 = g_v[t, pl.ds(s * 16, 16)]
                ir = lax.shift_right_logical(g16, 11)
                ic = lax.bitwise_and(g16, 2047)
                g_v[t, pl.ds(s * 16, 16)] = plsc.load_gather(row_v, [ir, ic])
        pltpu.sync_copy(g_v, lxz_hbm.at[b])


def _sc_gather_call(ftp, g2):
    mesh = plsc.VectorSubcoreMesh(core_axis_name="c", subcore_axis_name="s")
    return pl.kernel(
        _sc_gather,
        out_type=jax.ShapeDtypeStruct((BSZ, NCH, 128), jnp.int32),
        mesh=mesh,
        scratch_types=[
            pltpu.VMEM((NBLK, RB), jnp.int32),
            pltpu.VMEM((NCH, 128), jnp.int32),
            pltpu.SemaphoreType.DMA,
        ],
        compiler_params=pltpu.CompilerParams(needs_layout_passes=False),
    )(ftp, g2)


def _upd_body(y_ref, w_ref, mem_ref, x_ref, out_ref, u_ref, my_ref,
              csem, rsem, wsem):
    big = pltpu.make_async_copy(mem_ref, out_ref, csem)
    big.start()

    def rd(b, _):
        pltpu.make_async_copy(
            mem_ref.at[pl.ds(y_ref[b], 1)], my_ref.at[pl.ds(b, 1)],
            rsem).start()
        return 0

    lax.fori_loop(0, BSZ, rd, 0)

    def rdrain(b, _):
        pltpu.make_async_copy(
            mem_ref.at[pl.ds(0, 1)], my_ref.at[pl.ds(0, 1)], rsem).wait()
        return 0

    lax.fori_loop(0, BSZ, rdrain, 0)
    u = my_ref[...] * M + x_ref[...] * (1.0 - M)
    nrm = jnp.sqrt(jnp.sum(u * u, axis=1, keepdims=True))
    u_ref[...] = u / jnp.maximum(nrm, EPS)
    big.wait()

    def issue(b, _):
        pltpu.make_async_copy(
            u_ref.at[pl.ds(w_ref[b], 1)], out_ref.at[pl.ds(y_ref[b], 1)],
            wsem).start()
        return 0

    lax.fori_loop(0, BSZ, issue, 0)

    def drain(b, _):
        pltpu.make_async_copy(
            u_ref.at[pl.ds(0, 1)], out_ref.at[pl.ds(0, 1)], wsem).wait()
        return 0

    lax.fori_loop(0, BSZ, drain, 0)


def _update_call(y32, wv, memory, x):
    return pl.pallas_call(
        _upd_body,
        in_specs=[
            pl.BlockSpec(memory_space=pltpu.SMEM),
            pl.BlockSpec(memory_space=pltpu.SMEM),
            pl.BlockSpec(memory_space=pltpu.HBM),
            pl.BlockSpec(memory_space=pltpu.VMEM),
        ],
        out_specs=pl.BlockSpec(memory_space=pltpu.HBM),
        out_shape=jax.ShapeDtypeStruct((N_DATA, N_DIM), jnp.float32),
        scratch_shapes=[
            pltpu.VMEM((BSZ, N_DIM), jnp.float32),
            pltpu.VMEM((BSZ, N_DIM), jnp.float32),
            pltpu.SemaphoreType.DMA,
            pltpu.SemaphoreType.DMA,
            pltpu.SemaphoreType.DMA,
        ],
    )(y32, wv, memory, x)


def kernel(x, z, y, memory, idx):
    y32 = y.astype(jnp.int32)
    g = jnp.concatenate([y32[:, None], idx.astype(jnp.int32)], axis=1)
    g = jnp.pad(g, ((0, 0), (0, PADW - (K + 1))))
    g2 = g.reshape(BSZ, NCH, 128)
    xz = (jnp.concatenate([x, z], axis=0) / T).astype(jnp.bfloat16)
    ftp = _logits_all(memory, xz)
    lxz = _sc_gather_call(ftp, g2)
    lxz = lxz.reshape(BSZ, PADW)
    lx = lax.bitcast_convert_type(lxz << 16, jnp.float32)[:, :K + 1]
    lz = lax.bitcast_convert_type(lxz & (-65536), jnp.float32)[:, :K + 1]
    eq = y32[:, None] == y32[None, :]
    wv = jnp.max(
        jnp.where(eq, jnp.arange(BSZ, dtype=jnp.int32)[None, :], -1), axis=1)
    new_memory = _update_call(y32, wv, memory, x)
    return lx, lz, new_memory


# speedup vs baseline: 8.3803x; 8.3803x over previous
import functools

import jax
import jax.numpy as jnp
from jax import lax
from jax.experimental import pallas as pl
from jax.experimental.pallas import tpu as pltpu
from jax.experimental.pallas import tpu_sc as plsc

N_DATA = 100000
N_DIM = 128
BSZ = 256
K = 2048
T = 0.07
M = 0.5
EPS = 1e-12

RB = 2048
NBLK = 49
NPAD = RB * NBLK
SLAB = NPAD // 128
NW = 32
BPW = BSZ // NW
NCH = 24
PADW = NCH * 128


def _mm_body(xz_ref, mem_ref, f_ref):
    a = mem_ref[...]
    d = lax.dot_general(xz_ref[...], a.astype(jnp.bfloat16),
                        (((1,), (1,)), ((), ())),
                        preferred_element_type=jnp.float32)
    def rne(v32):
        return (v32 + 0x7FFF + ((v32 >> 16) & 1)) & jnp.uint32(0xFFFF0000)

    xb = rne(lax.bitcast_convert_type(d[:BSZ], jnp.uint32))
    zb = rne(lax.bitcast_convert_type(d[BSZ:], jnp.uint32))
    packed = lax.bitcast_convert_type((xb >> 16) | zb, jnp.int32)
    f_ref[0] = packed


def _logits_all(memory, xz):
    return pl.pallas_call(
        _mm_body,
        grid=(NBLK,),
        in_specs=[
            pl.BlockSpec((2 * BSZ, N_DIM), lambda i: (0, 0)),
            pl.BlockSpec((RB, N_DIM), lambda i: (i, 0)),
        ],
        out_specs=pl.BlockSpec((1, BSZ, RB), lambda i: (i, 0, 0)),
        out_shape=jax.ShapeDtypeStruct((NBLK, BSZ, RB), jnp.int32),
    )(xz, memory)


def _sc_gather(ftp_hbm, g_hbm, y_hbm, mem_hbm, lxz_hbm, my_hbm,
               row_v, g_v, y_v, my_v, rsem, ssem):
    wid = lax.axis_index("s") * 2 + lax.axis_index("c")
    b0 = wid * BPW
    pltpu.sync_copy(y_hbm.at[pl.ds(b0, BPW)], y_v)
    row_cp = pltpu.async_copy(mem_hbm.at[y_v], my_v, rsem)

    @pl.loop(0, BPW)
    def _per_b(lb):
        b = b0 + lb
        cp1 = pltpu.async_copy(
            ftp_hbm.at[:, b, pl.ds(0, RB // 2)],
            row_v.at[:, pl.ds(0, RB // 2)], ssem)
        cp2 = pltpu.async_copy(
            ftp_hbm.at[:, b, pl.ds(RB // 2, RB // 2)],
            row_v.at[:, pl.ds(RB // 2, RB // 2)], ssem)
        pltpu.sync_copy(g_hbm.at[b], g_v)
        cp1.wait()
        cp2.wait()
        for t in range(NCH):
            for s in range(8):
                g16 = g_v[t, pl.ds(s * 16, 16)]
                ir = lax.shift_right_logical(g16, 11)
                ic = lax.bitwise_and(g16, 2047)
                g_v[t, pl.ds(s * 16, 16)] = plsc.load_gather(row_v, [ir, ic])
        pltpu.sync_copy(g_v, lxz_hbm.at[b])

    row_cp.wait()
    pltpu.sync_copy(my_v, my_hbm.at[pl.ds(b0, BPW)])


def _sc_gather_call(ftp, g2, y32, memory):
    mesh = plsc.VectorSubcoreMesh(core_axis_name="c", subcore_axis_name="s")
    return pl.kernel(
        _sc_gather,
        out_type=(
            jax.ShapeDtypeStruct((BSZ, NCH, 128), jnp.int32),
            jax.ShapeDtypeStruct((BSZ, N_DIM), jnp.float32),
        ),
        mesh=mesh,
        scratch_types=[
            pltpu.VMEM((NBLK, RB), jnp.int32),
            pltpu.VMEM((NCH, 128), jnp.int32),
            pltpu.VMEM((BPW,), jnp.int32),
            pltpu.VMEM((BPW, N_DIM), jnp.float32),
            pltpu.SemaphoreType.DMA,
            pltpu.SemaphoreType.DMA,
        ],
        compiler_params=pltpu.CompilerParams(needs_layout_passes=False),
    )(ftp, g2, y32, memory)


def _copy_body(mem_ref, out_ref):
    out_ref[...] = mem_ref[...]


def _copy_call(memory):
    return pl.pallas_call(
        _copy_body,
        grid=(25,),
        in_specs=[pl.BlockSpec((4000, N_DIM), lambda i: (i, 0))],
        out_specs=pl.BlockSpec((4000, N_DIM), lambda i: (i, 0)),
        out_shape=jax.ShapeDtypeStruct((N_DATA, N_DIM), jnp.float32),
    )(memory)


def _upd_body(y_ref, w_ref, base_ref, my_ref, x_ref, out_ref, u_ref, sem):
    del base_ref
    u = my_ref[...] * M + x_ref[...] * (1.0 - M)
    nrm = jnp.sqrt(jnp.sum(u * u, axis=1, keepdims=True))
    u_ref[...] = u / jnp.maximum(nrm, EPS)

    def issue(b, _):
        pltpu.make_async_copy(
            u_ref.at[pl.ds(w_ref[b], 1)], out_ref.at[pl.ds(y_ref[b], 1)],
            sem).start()
        return 0

    lax.fori_loop(0, BSZ, issue, 0)

    def drain(b, _):
        pltpu.make_async_copy(
            u_ref.at[pl.ds(0, 1)], out_ref.at[pl.ds(0, 1)], sem).wait()
        return 0

    lax.fori_loop(0, BSZ, drain, 0)


def _update_call(y32, wv, base, my, x):
    return pl.pallas_call(
        _upd_body,
        in_specs=[
            pl.BlockSpec(memory_space=pltpu.SMEM),
            pl.BlockSpec(memory_space=pltpu.SMEM),
            pl.BlockSpec(memory_space=pltpu.HBM),
            pl.BlockSpec(memory_space=pltpu.VMEM),
            pl.BlockSpec(memory_space=pltpu.VMEM),
        ],
        out_specs=pl.BlockSpec(memory_space=pltpu.HBM),
        out_shape=jax.ShapeDtypeStruct((N_DATA, N_DIM), jnp.float32),
        scratch_shapes=[
            pltpu.VMEM((BSZ, N_DIM), jnp.float32),
            pltpu.SemaphoreType.DMA,
        ],
        input_output_aliases={2: 0},
    )(y32, wv, base, my, x)


def kernel(x, z, y, memory, idx):
    y32 = y.astype(jnp.int32)
    g = jnp.concatenate([y32[:, None], idx.astype(jnp.int32)], axis=1)
    g = jnp.pad(g, ((0, 0), (0, PADW - (K + 1))))
    g2 = g.reshape(BSZ, NCH, 128)
    xz = (jnp.concatenate([x, z], axis=0) / T).astype(jnp.bfloat16)
    ftp = _logits_all(memory, xz)
    base = _copy_call(memory)
    lxz, my = _sc_gather_call(ftp, g2, y32, memory)
    lxz = lxz.reshape(BSZ, PADW)
    lx = lax.bitcast_convert_type(lxz << 16, jnp.float32)[:, :K + 1]
    lz = lax.bitcast_convert_type(lxz & (-65536), jnp.float32)[:, :K + 1]
    eq = y32[:, None] == y32[None, :]
    wv = jnp.max(
        jnp.where(eq, jnp.arange(BSZ, dtype=jnp.int32)[None, :], -1), axis=1)
    new_memory = _update_call(y32, wv, base, my, x)
    return lx, lz, new_memory


# speedup vs baseline: 9.0886x; 1.0845x over previous
import jax
import jax.numpy as jnp
from jax import lax
from jax.experimental import pallas as pl
from jax.experimental.pallas import tpu as pltpu
from jax.experimental.pallas import tpu_sc as plsc

N_DATA = 100000
N_DIM = 128
BSZ = 256
K = 2048
T = 0.07
M = 0.5
EPS = 1e-12

RB = 2048
NBLK = 49
NPAD = RB * NBLK
SLAB = NPAD // 128
NW = 32
BPW = BSZ // NW
NCH = 24
PADW = NCH * 128


def _mm_body(xz_ref, mem_ref, f_ref, base_ref):
    a = mem_ref[...]
    d = lax.dot_general(xz_ref[...], a.astype(jnp.bfloat16),
                        (((1,), (1,)), ((), ())),
                        preferred_element_type=jnp.float32)
    def rne(v32):
        return (v32 + 0x7FFF + ((v32 >> 16) & 1)) & jnp.uint32(0xFFFF0000)

    xb = rne(lax.bitcast_convert_type(d[:BSZ], jnp.uint32))
    zb = rne(lax.bitcast_convert_type(d[BSZ:], jnp.uint32))
    packed = lax.bitcast_convert_type((xb >> 16) | zb, jnp.int32)
    f_ref[0] = packed
    base_ref[...] = a


def _logits_all(memory, xz):
    return pl.pallas_call(
        _mm_body,
        grid=(NBLK,),
        in_specs=[
            pl.BlockSpec((2 * BSZ, N_DIM), lambda i: (0, 0)),
            pl.BlockSpec((RB, N_DIM), lambda i: (i, 0)),
        ],
        out_specs=[
            pl.BlockSpec((1, BSZ, RB), lambda i: (i, 0, 0)),
            pl.BlockSpec((RB, N_DIM), lambda i: (i, 0)),
        ],
        out_shape=[
            jax.ShapeDtypeStruct((NBLK, BSZ, RB), jnp.int32),
            jax.ShapeDtypeStruct((N_DATA, N_DIM), jnp.float32),
        ],
    )(xz, memory)


def _sc_gather(ftp_hbm, g_hbm, y_hbm, mem_hbm, lxz_hbm, my_hbm,
               row_v, g_v, y_v, my_v, rsem, ssem):
    wid = lax.axis_index("s") * 2 + lax.axis_index("c")
    b0 = wid * BPW
    pltpu.sync_copy(y_hbm.at[pl.ds(b0, BPW)], y_v)
    row_cp = pltpu.async_copy(mem_hbm.at[y_v], my_v, rsem)

    @pl.loop(0, BPW)
    def _per_b(lb):
        b = b0 + lb
        cp1 = pltpu.async_copy(
            ftp_hbm.at[:, b, pl.ds(0, RB // 2)],
            row_v.at[:, pl.ds(0, RB // 2)], ssem)
        cp2 = pltpu.async_copy(
            ftp_hbm.at[:, b, pl.ds(RB // 2, RB // 2)],
            row_v.at[:, pl.ds(RB // 2, RB // 2)], ssem)
        pltpu.sync_copy(g_hbm.at[b], g_v)
        cp1.wait()
        cp2.wait()
        for t in range(NCH):
            for s in range(8):
                g16 = g_v[t, pl.ds(s * 16, 16)]
                ir = lax.shift_right_logical(g16, 11)
                ic = lax.bitwise_and(g16, 2047)
                g_v[t, pl.ds(s * 16, 16)] = plsc.load_gather(row_v, [ir, ic])
        pltpu.sync_copy(g_v, lxz_hbm.at[b])

    row_cp.wait()
    pltpu.sync_copy(my_v, my_hbm.at[pl.ds(b0, BPW)])


def _sc_gather_call(ftp, g2, y32, memory):
    mesh = plsc.VectorSubcoreMesh(core_axis_name="c", subcore_axis_name="s")
    return pl.kernel(
        _sc_gather,
        out_type=(
            jax.ShapeDtypeStruct((BSZ, NCH, 128), jnp.int32),
            jax.ShapeDtypeStruct((BSZ, N_DIM), jnp.float32),
        ),
        mesh=mesh,
        scratch_types=[
            pltpu.VMEM((NBLK, RB), jnp.int32),
            pltpu.VMEM((NCH, 128), jnp.int32),
            pltpu.VMEM((BPW,), jnp.int32),
            pltpu.VMEM((BPW, N_DIM), jnp.float32),
            pltpu.SemaphoreType.DMA,
            pltpu.SemaphoreType.DMA,
        ],
        compiler_params=pltpu.CompilerParams(needs_layout_passes=False),
    )(ftp, g2, y32, memory)


def _upd_body(y_ref, w_ref, base_ref, my_ref, x_ref, out_ref, u_ref, sem):
    del base_ref
    u = my_ref[...] * M + x_ref[...] * (1.0 - M)
    nrm = jnp.sqrt(jnp.sum(u * u, axis=1, keepdims=True))
    u_ref[...] = u / jnp.maximum(nrm, EPS)

    def issue(b, _):
        pltpu.make_async_copy(
            u_ref.at[pl.ds(w_ref[b], 1)], out_ref.at[pl.ds(y_ref[b], 1)],
            sem).start()
        return 0

    lax.fori_loop(0, BSZ, issue, 0)

    def drain(b, _):
        pltpu.make_async_copy(
            u_ref.at[pl.ds(0, 1)], out_ref.at[pl.ds(0, 1)], sem).wait()
        return 0

    lax.fori_loop(0, BSZ, drain, 0)


def _update_call(y32, wv, base, my, x):
    return pl.pallas_call(
        _upd_body,
        in_specs=[
            pl.BlockSpec(memory_space=pltpu.SMEM),
            pl.BlockSpec(memory_space=pltpu.SMEM),
            pl.BlockSpec(memory_space=pltpu.HBM),
            pl.BlockSpec(memory_space=pltpu.VMEM),
            pl.BlockSpec(memory_space=pltpu.VMEM),
        ],
        out_specs=pl.BlockSpec(memory_space=pltpu.HBM),
        out_shape=jax.ShapeDtypeStruct((N_DATA, N_DIM), jnp.float32),
        scratch_shapes=[
            pltpu.VMEM((BSZ, N_DIM), jnp.float32),
            pltpu.SemaphoreType.DMA,
        ],
        input_output_aliases={2: 0},
    )(y32, wv, base, my, x)


def kernel(x, z, y, memory, idx):
    y32 = y.astype(jnp.int32)
    g = jnp.concatenate([y32[:, None], idx.astype(jnp.int32)], axis=1)
    g = jnp.pad(g, ((0, 0), (0, PADW - (K + 1))))
    g2 = g.reshape(BSZ, NCH, 128)
    xz = (jnp.concatenate([x, z], axis=0) / T).astype(jnp.bfloat16)
    ftp, base = _logits_all(memory, xz)
    lxz, my = _sc_gather_call(ftp, g2, y32, memory)
    lxz = lxz.reshape(BSZ, PADW)
    lx = lax.bitcast_convert_type(lxz << 16, jnp.float32)[:, :K + 1]
    lz = lax.bitcast_convert_type(lxz & (-65536), jnp.float32)[:, :K + 1]
    eq = y32[:, None] == y32[None, :]
    wv = jnp.max(
        jnp.where(eq, jnp.arange(BSZ, dtype=jnp.int32)[None, :], -1), axis=1)
    new_memory = _update_call(y32, wv, base, my, x)
    return lx, lz, new_memory
